# Initial kernel scaffold; baseline (speedup 1.0000x reference)
import jax
import jax.numpy as jnp
from jax.experimental import pallas as pl


def kernel(words, table, TI):
    raise NotImplementedError("write your pallas kernel here")


# speedup vs baseline: 2.9541x; 2.9541x over previous
import functools

import jax
import jax.numpy as jnp
from jax import lax
from jax.experimental import pallas as pl
from jax.experimental.pallas import tpu as pltpu
from jax.experimental.pallas import tpu_sc as plsc

NC = 2
NS = 16
NW = NC * NS
LANE = 16


def _sent_vec_tfidf(words_flat, table, TI, B, L):
    V, D = table.shape
    RB = B // NW
    CH = 32
    CL = CH * L
    NCHUNK = RB // CH
    NBUF = 2

    mesh = plsc.VectorSubcoreMesh(core_axis_name="c", subcore_axis_name="s")

    @functools.partial(
        pl.kernel,
        out_type=jax.ShapeDtypeStruct((B, D), jnp.float32),
        mesh=mesh,
        compiler_params=pltpu.CompilerParams(use_tc_tiling_on_sc=False),
        scratch_types=dict(
            idx_v=[pltpu.VMEM((CL,), jnp.int32) for _ in range(NBUF)],
            rows_v=[pltpu.VMEM((CL, D), jnp.float32) for _ in range(NBUF)],
            tiv_v=[pltpu.VMEM((CL,), jnp.float32) for _ in range(NBUF)],
            outs_v=[pltpu.VMEM((CH, D), jnp.float32) for _ in range(NBUF)],
            rsem=[pltpu.SemaphoreType.DMA for _ in range(NBUF)],
            tsem=[pltpu.SemaphoreType.DMA for _ in range(NBUF)],
            osem=[pltpu.SemaphoreType.DMA for _ in range(NBUF)],
        ),
    )
    def k(words_hbm, table_hbm, ti_hbm, out_hbm, *, idx_v, rows_v, tiv_v,
          outs_v, rsem, tsem, osem):
        wid = lax.axis_index("s") * NC + lax.axis_index("c")
        row0 = wid * RB

        def issue(c, p):
            base = (row0 + c * CH) * L
            pltpu.sync_copy(words_hbm.at[pl.ds(base, CL)], idx_v[p])
            pltpu.async_copy(table_hbm.at[idx_v[p]], rows_v[p], rsem[p])
            pltpu.async_copy(ti_hbm.at[idx_v[p]], tiv_v[p], tsem[p])

        def wait_gathers(p):
            pltpu.make_async_copy(table_hbm.at[idx_v[p]], rows_v[p],
                                  rsem[p]).wait()
            pltpu.make_async_copy(ti_hbm.at[idx_v[p]], tiv_v[p],
                                  tsem[p]).wait()

        lane_iota = lax.iota(jnp.int32, LANE)

        def compute(p):
            tiv = tiv_v[p]
            rows = rows_v[p]
            outs = outs_v[p]

            def row_body(r, _):
                rl = r * L
                w0 = tiv[pl.ds(rl, LANE)]
                w1 = tiv[pl.ds(rl + 16, LANE)]
                w2 = tiv[pl.ds(rl + 32, LANE)]
                w3 = tiv[pl.ds(rl + L - LANE, LANE)]
                w3m = jnp.where(lane_iota >= (48 - (L - LANE)), w3, 0.0)
                wv = w0 + w1 + w2 + w3m
                for sh in (1, 2, 4, 8):
                    wv = wv + jnp.take_along_axis(
                        wv, jnp.bitwise_xor(lane_iota, sh), axis=0,
                        mode="promise_in_bounds")
                inv = 1.0 / (wv + 1e-8)

                chunks = (w0, w1, w2, w3)
                acc0 = jnp.zeros((LANE,), jnp.float32)
                acc1 = jnp.zeros((LANE,), jnp.float32)
                for l in range(L):
                    if l < 48:
                        cidx, lane = l // LANE, l % LANE
                    else:
                        cidx, lane = 3, l - (L - LANE)
                    wl = jnp.take_along_axis(
                        chunks[cidx], jnp.full((LANE,), lane, jnp.int32),
                        axis=0, mode="promise_in_bounds")
                    r0 = rows[rl + l, pl.ds(0, LANE)]
                    r1 = rows[rl + l, pl.ds(LANE, LANE)]
                    acc0 = acc0 + wl * r0
                    acc1 = acc1 + wl * r1
                outs[r, pl.ds(0, LANE)] = acc0 * inv
                outs[r, pl.ds(LANE, LANE)] = acc1 * inv
                return 0

            lax.fori_loop(0, CH, row_body, 0)

        for p in range(min(NBUF, NCHUNK)):
            issue(p, p)

        for c in range(NCHUNK):
            p = c % NBUF
            base = row0 + c * CH
            wait_gathers(p)
            if c >= NBUF:
                pltpu.make_async_copy(
                    outs_v[p], out_hbm.at[pl.ds(base - NBUF * CH, CH)],
                    osem[p]).wait()
            compute(p)
            pltpu.async_copy(outs_v[p], out_hbm.at[pl.ds(base, CH)], osem[p])
            nxt = c + NBUF
            if nxt < NCHUNK:
                issue(nxt, p)

        for c in range(max(0, NCHUNK - NBUF), NCHUNK):
            p = c % NBUF
            base = row0 + c * CH
            pltpu.make_async_copy(outs_v[p], out_hbm.at[pl.ds(base, CH)],
                                  osem[p]).wait()

    return k(words_flat, table, TI)


def kernel(words, table, TI):
    B, L = words.shape
    words_flat = words.astype(jnp.int32).reshape(-1)
    return _sent_vec_tfidf(words_flat, table, TI, B, L)
